# Initial kernel scaffold; baseline (speedup 1.0000x reference)
import jax
import jax.numpy as jnp
from jax.experimental import pallas as pl


def kernel(embeddings, pos_table):
    raise NotImplementedError("write your pallas kernel here")


# speedup vs baseline: 1.6901x; 1.6901x over previous
import jax
import jax.numpy as jnp
from jax.experimental import pallas as pl


_TT = 256


def _add_kernel(emb_ref, pos_ref, out_ref):
    out_ref[...] = emb_ref[...] + pos_ref[...]


def kernel(embeddings, pos_table):
    Bn, Tn, Hn = embeddings.shape
    tt = _TT if Tn % _TT == 0 else Tn
    grid = (Tn // tt, Bn)
    return pl.pallas_call(
        _add_kernel,
        grid=grid,
        in_specs=[
            pl.BlockSpec((1, tt, Hn), lambda t, b: (b, t, 0)),
            pl.BlockSpec((tt, Hn), lambda t, b: (t, 0)),
        ],
        out_specs=pl.BlockSpec((1, tt, Hn), lambda t, b: (b, t, 0)),
        out_shape=jax.ShapeDtypeStruct((Bn, Tn, Hn), embeddings.dtype),
    )(embeddings, pos_table)


# speedup vs baseline: 1.8822x; 1.1137x over previous
import jax
import jax.numpy as jnp
from jax.experimental import pallas as pl


_TT = 512


def _add_kernel(emb_ref, pos_ref, out_ref):
    out_ref[...] = emb_ref[...] + pos_ref[...]


def kernel(embeddings, pos_table):
    Bn, Tn, Hn = embeddings.shape
    tt = _TT if Tn % _TT == 0 else Tn
    grid = (Tn // tt, Bn)
    return pl.pallas_call(
        _add_kernel,
        grid=grid,
        in_specs=[
            pl.BlockSpec((1, tt, Hn), lambda t, b: (b, t, 0)),
            pl.BlockSpec((tt, Hn), lambda t, b: (t, 0)),
        ],
        out_specs=pl.BlockSpec((1, tt, Hn), lambda t, b: (b, t, 0)),
        out_shape=jax.ShapeDtypeStruct((Bn, Tn, Hn), embeddings.dtype),
    )(embeddings, pos_table)


# speedup vs baseline: 1.9873x; 1.0558x over previous
import jax
import jax.numpy as jnp
from jax.experimental import pallas as pl


_TT = 1024


def _add_kernel(emb_ref, pos_ref, out_ref):
    out_ref[...] = emb_ref[...] + pos_ref[...]


def kernel(embeddings, pos_table):
    Bn, Tn, Hn = embeddings.shape
    tt = _TT if Tn % _TT == 0 else Tn
    grid = (Tn // tt, Bn)
    return pl.pallas_call(
        _add_kernel,
        grid=grid,
        in_specs=[
            pl.BlockSpec((1, tt, Hn), lambda t, b: (b, t, 0)),
            pl.BlockSpec((tt, Hn), lambda t, b: (t, 0)),
        ],
        out_specs=pl.BlockSpec((1, tt, Hn), lambda t, b: (b, t, 0)),
        out_shape=jax.ShapeDtypeStruct((Bn, Tn, Hn), embeddings.dtype),
    )(embeddings, pos_table)
